# Initial kernel scaffold; baseline (speedup 1.0000x reference)
import jax
import jax.numpy as jnp
from jax.experimental import pallas as pl


def kernel(x, edge_index, W1, b1, W2, b2):
    raise NotImplementedError("write your pallas kernel here")


# speedup vs baseline: 10.4858x; 10.4858x over previous
import functools

import jax
import jax.numpy as jnp
from jax import lax
from jax.experimental import pallas as pl
from jax.experimental.pallas import tpu as pltpu
from jax.experimental.pallas import tpu_sc as plsc

N = 10000
E = 320000
NP = 10240
EP = 327680
CHUNK = 128
NCHUNKS = EP // CHUNK
NC, NS = 2, 16
NW = NC * NS
CPW = NCHUNKS // NW
TROWS = NP // NS

_MESH = dict(core_axis_name="c", subcore_axis_name="s", num_cores=NC,
             num_subcores=NS)


def _sc_scatter():
    @functools.partial(
        pl.kernel,
        out_type=jax.ShapeDtypeStruct((NC, NP, 128), jnp.float32),
        mesh=plsc.VectorSubcoreMesh(**_MESH),
        scratch_types=[
            pltpu.VMEM((2, 2, CHUNK), jnp.int32),
            pltpu.VMEM((2, CHUNK, 128), jnp.float32),
            pltpu.VMEM_SHARED((NP, 128), jnp.float32),
            pltpu.SemaphoreType.DMA,
            pltpu.SemaphoreType.DMA,
            pltpu.SemaphoreType.DMA,
            pltpu.SemaphoreType.DMA,
        ],
    )
    def k(h_hbm, sd_hbm, zeros_hbm, out_hbm,
          ib, rows, acc, si0, si1, sg0, sg1):
        c = lax.axis_index("c")
        t = lax.axis_index("s")
        wid = c * NS + t
        base = wid * CPW
        pltpu.sync_copy(zeros_hbm, rows.at[0])
        for z in range(TROWS // CHUNK):
            pltpu.sync_copy(rows.at[0],
                            acc.at[pl.ds(t * TROWS + z * CHUNK, CHUNK)])
        pltpu.async_copy(sd_hbm.at[base], ib.at[0], si0)
        pltpu.make_async_copy(sd_hbm.at[0], ib.at[0], si0).wait()
        pltpu.async_copy(h_hbm.at[ib.at[0].at[0]], rows.at[0], sg0)
        pltpu.async_copy(sd_hbm.at[base + 1], ib.at[1], si1)
        plsc.subcore_barrier()

        def half(j, b, nb, sib, sinb, sgb, sgnb):
            @pl.when(j + 1 < CPW)
            def _():
                pltpu.make_async_copy(sd_hbm.at[0], ib.at[nb], sinb).wait()
                pltpu.async_copy(h_hbm.at[ib.at[nb].at[0]], rows.at[nb],
                                 sgnb)
            pltpu.make_async_copy(h_hbm.at[pl.ds(0, CHUNK)], rows.at[b],
                                  sgb).wait()
            pltpu.sync_copy(rows.at[b], acc.at[ib.at[b].at[1]], add=True)

            @pl.when(j + 2 < CPW)
            def _():
                pltpu.async_copy(sd_hbm.at[base + j + 2], ib.at[b], sib)

        def step(i, _):
            j = 2 * i
            half(j, 0, 1, si0, si1, sg0, sg1)
            half(j + 1, 1, 0, si1, si0, sg1, sg0)
            return 0

        lax.fori_loop(0, CPW // 2, step, 0)
        plsc.subcore_barrier()
        pltpu.sync_copy(acc.at[pl.ds(t * TROWS, TROWS)],
                        out_hbm.at[c].at[pl.ds(t * TROWS, TROWS)])

    return k


def _sc_degree():
    @functools.partial(
        pl.kernel,
        out_type=jax.ShapeDtypeStruct((NC, NP, 128), jnp.float32),
        mesh=plsc.VectorSubcoreMesh(**_MESH),
        scratch_types=[
            pltpu.VMEM((2, 2, CHUNK), jnp.int32),
            pltpu.VMEM((CHUNK, 128), jnp.float32),
            pltpu.VMEM((CHUNK, 128), jnp.float32),
            pltpu.VMEM_SHARED((NP, 128), jnp.float32),
            pltpu.SemaphoreType.DMA,
            pltpu.SemaphoreType.DMA,
        ],
    )
    def k(sd_hbm, ones_hbm, zeros_hbm, out_hbm, ib, onesv, zb, acc,
          si0, si1):
        c = lax.axis_index("c")
        t = lax.axis_index("s")
        wid = c * NS + t
        base = wid * CPW
        pltpu.sync_copy(zeros_hbm, zb)
        for z in range(TROWS // CHUNK):
            pltpu.sync_copy(zb, acc.at[pl.ds(t * TROWS + z * CHUNK, CHUNK)])
        pltpu.sync_copy(ones_hbm, onesv)
        pltpu.async_copy(sd_hbm.at[base], ib.at[0], si0)
        pltpu.async_copy(sd_hbm.at[base + 1], ib.at[1], si1)
        plsc.subcore_barrier()

        def half(j, b, sib):
            pltpu.make_async_copy(sd_hbm.at[0], ib.at[b], sib).wait()
            pltpu.sync_copy(onesv, acc.at[ib.at[b].at[1]], add=True)

            @pl.when(j + 2 < CPW)
            def _():
                pltpu.async_copy(sd_hbm.at[base + j + 2], ib.at[b], sib)

        def step(i, _):
            j = 2 * i
            half(j, 0, si0)
            half(j + 1, 1, si1)
            return 0

        lax.fori_loop(0, CPW // 2, step, 0)
        plsc.subcore_barrier()
        pltpu.sync_copy(acc.at[pl.ds(t * TROWS, TROWS)],
                        out_hbm.at[c].at[pl.ds(t * TROWS, TROWS)])

    return k


_ROWS_B = 1024
_GRID = NP // _ROWS_B


def _dinv(deg_ref):
    return lax.rsqrt(deg_ref[0, :, 0:1] + deg_ref[1, :, 0:1] + 1.0)


def _tc1_body(x_ref, w_ref, deg_ref, o_ref):
    dinv = _dinv(deg_ref)
    o_ref[...] = jnp.dot(x_ref[...], w_ref[...],
                         preferred_element_type=jnp.float32) * dinv


def _tc2_body(a_ref, h_ref, deg_ref, b_ref, w_ref, o_ref):
    dinv = _dinv(deg_ref)
    z = jnp.maximum(dinv * (a_ref[0] + a_ref[1] + h_ref[...]) + b_ref[...],
                    0.0)
    o_ref[...] = jnp.dot(z, w_ref[...],
                         preferred_element_type=jnp.float32) * dinv


def _tc3_body(a_ref, h_ref, deg_ref, b_ref, o_ref):
    dinv = _dinv(deg_ref)
    agg = (a_ref[0] + a_ref[1] + h_ref[...])[:, :64]
    o_ref[...] = jnp.maximum(dinv * agg + b_ref[...], 0.0)


def _rows_spec(fw):
    return pl.BlockSpec((_ROWS_B, fw), lambda i: (i, 0))


def _part_spec(fw):
    return pl.BlockSpec((NC, _ROWS_B, fw), lambda i: (0, i, 0))


def _full_spec(a, b):
    return pl.BlockSpec((a, b), lambda i: (0, 0))


def kernel(x, edge_index, W1, b1, W2, b2):
    s = edge_index[0].astype(jnp.int32)
    d = edge_index[1].astype(jnp.int32)
    pad = EP - E
    s2 = jnp.concatenate([s, jnp.zeros((pad,), jnp.int32)]).reshape(
        NCHUNKS, CHUNK)
    d2 = jnp.concatenate([d, jnp.full((pad,), N, jnp.int32)]).reshape(
        NCHUNKS, CHUNK)
    sd2 = jnp.stack([s2, d2], axis=1)
    xp = jnp.pad(x, ((0, NP - N), (0, 0)))
    W2p = jnp.pad(W2, ((0, 0), (0, 64)))
    ones128 = jnp.ones((CHUNK, 128), jnp.float32)
    z128 = jnp.zeros((CHUNK, 128), jnp.float32)

    degp = _sc_degree()(sd2, ones128, z128)

    h1s = pl.pallas_call(
        _tc1_body,
        grid=(_GRID,),
        in_specs=[_rows_spec(128), _full_spec(128, 128), _part_spec(128)],
        out_specs=_rows_spec(128),
        out_shape=jax.ShapeDtypeStruct((NP, 128), jnp.float32),
    )(xp, W1, degp)

    sc_scatter = _sc_scatter()
    a1 = sc_scatter(h1s, sd2, z128)

    h2s = pl.pallas_call(
        _tc2_body,
        grid=(_GRID,),
        in_specs=[_part_spec(128), _rows_spec(128), _part_spec(128),
                  _full_spec(1, 128), _full_spec(128, 128)],
        out_specs=_rows_spec(128),
        out_shape=jax.ShapeDtypeStruct((NP, 128), jnp.float32),
    )(a1, h1s, degp, b1.reshape(1, 128), W2p)

    a2 = sc_scatter(h2s, sd2, z128)

    outp = pl.pallas_call(
        _tc3_body,
        grid=(_GRID,),
        in_specs=[_part_spec(128), _rows_spec(128), _part_spec(128),
                  _full_spec(1, 64)],
        out_specs=_rows_spec(64),
        out_shape=jax.ShapeDtypeStruct((NP, 64), jnp.float32),
    )(a2, h2s, degp, b2.reshape(1, 64))

    return outp[:N]


# speedup vs baseline: 24.8312x; 2.3681x over previous
import functools

import jax
import jax.numpy as jnp
from jax import lax
from jax.experimental import pallas as pl
from jax.experimental.pallas import tpu as pltpu
from jax.experimental.pallas import tpu_sc as plsc

N = 10000
E = 320000
NP = 10240
EP = 327680
CHUNK = 128
NCHUNKS = EP // CHUNK
NC, NS = 2, 16
NW = NC * NS
CPW = NCHUNKS // NW
TROWS = NP // NS

_MESH = dict(core_axis_name="c", subcore_axis_name="s", num_cores=NC,
             num_subcores=NS)


def _sc_scatter():
    @functools.partial(
        pl.kernel,
        out_type=jax.ShapeDtypeStruct((NC, NP, 128), jnp.float32),
        mesh=plsc.VectorSubcoreMesh(**_MESH),
        scratch_types=[
            pltpu.VMEM((2, 2, CHUNK), jnp.int32),
            pltpu.VMEM((2, CHUNK, 128), jnp.float32),
            pltpu.VMEM_SHARED((NP, 128), jnp.float32),
            pltpu.SemaphoreType.DMA,
            pltpu.SemaphoreType.DMA,
            pltpu.SemaphoreType.DMA,
            pltpu.SemaphoreType.DMA,
        ],
    )
    def k(h_hbm, sd_hbm, zeros_hbm, out_hbm,
          ib, rows, acc, si0, si1, sg0, sg1):
        c = lax.axis_index("c")
        t = lax.axis_index("s")
        wid = c * NS + t
        base = wid * CPW
        pltpu.sync_copy(zeros_hbm, rows.at[0])
        for z in range(TROWS // CHUNK):
            pltpu.sync_copy(rows.at[0],
                            acc.at[pl.ds(t * TROWS + z * CHUNK, CHUNK)])
        pltpu.async_copy(sd_hbm.at[base], ib.at[0], si0)
        pltpu.make_async_copy(sd_hbm.at[0], ib.at[0], si0).wait()
        pltpu.async_copy(h_hbm.at[ib.at[0].at[0]], rows.at[0], sg0)
        pltpu.async_copy(sd_hbm.at[base + 1], ib.at[1], si1)
        plsc.subcore_barrier()

        def half(j, b, nb, sib, sinb, sgb, sgnb):
            @pl.when(j + 1 < CPW)
            def _():
                pltpu.make_async_copy(sd_hbm.at[0], ib.at[nb], sinb).wait()
                pltpu.async_copy(h_hbm.at[ib.at[nb].at[0]], rows.at[nb],
                                 sgnb)
            pltpu.make_async_copy(h_hbm.at[pl.ds(0, CHUNK)], rows.at[b],
                                  sgb).wait()
            pltpu.sync_copy(rows.at[b], acc.at[ib.at[b].at[1]], add=True)

            @pl.when(j + 2 < CPW)
            def _():
                pltpu.async_copy(sd_hbm.at[base + j + 2], ib.at[b], sib)

        def step(i, _):
            j = 2 * i
            half(j, 0, 1, si0, si1, sg0, sg1)
            half(j + 1, 1, 0, si1, si0, sg1, sg0)
            return 0

        lax.fori_loop(0, CPW // 2, step, 0)
        plsc.subcore_barrier()
        pltpu.sync_copy(acc.at[pl.ds(t * TROWS, TROWS)],
                        out_hbm.at[c].at[pl.ds(t * TROWS, TROWS)])

    return k


def _sc_degree():
    @functools.partial(
        pl.kernel,
        out_type=jax.ShapeDtypeStruct((NC, NP, 128), jnp.float32),
        mesh=plsc.VectorSubcoreMesh(**_MESH),
        scratch_types=[
            pltpu.VMEM((2, 2, CHUNK), jnp.int32),
            pltpu.VMEM((CHUNK, 128), jnp.float32),
            pltpu.VMEM((CHUNK, 128), jnp.float32),
            pltpu.VMEM_SHARED((NP, 128), jnp.float32),
            pltpu.SemaphoreType.DMA,
            pltpu.SemaphoreType.DMA,
        ],
    )
    def k(sd_hbm, ones_hbm, zeros_hbm, out_hbm, ib, onesv, zb, acc,
          si0, si1):
        c = lax.axis_index("c")
        t = lax.axis_index("s")
        wid = c * NS + t
        base = wid * CPW
        pltpu.sync_copy(zeros_hbm, zb)
        for z in range(TROWS // CHUNK):
            pltpu.sync_copy(zb, acc.at[pl.ds(t * TROWS + z * CHUNK, CHUNK)])
        pltpu.sync_copy(ones_hbm, onesv)
        pltpu.async_copy(sd_hbm.at[base], ib.at[0], si0)
        pltpu.async_copy(sd_hbm.at[base + 1], ib.at[1], si1)
        plsc.subcore_barrier()

        def half(j, b, sib):
            pltpu.make_async_copy(sd_hbm.at[0], ib.at[b], sib).wait()
            pltpu.sync_copy(onesv, acc.at[ib.at[b].at[1]], add=True)

            @pl.when(j + 2 < CPW)
            def _():
                pltpu.async_copy(sd_hbm.at[base + j + 2], ib.at[b], sib)

        def step(i, _):
            j = 2 * i
            half(j, 0, si0)
            half(j + 1, 1, si1)
            return 0

        lax.fori_loop(0, CPW // 2, step, 0)
        plsc.subcore_barrier()
        pltpu.sync_copy(acc.at[pl.ds(t * TROWS, TROWS)],
                        out_hbm.at[c].at[pl.ds(t * TROWS, TROWS)])

    return k


_ROWS_B = 1024
_GRID = NP // _ROWS_B


def _dinv(deg_ref):
    return lax.rsqrt(deg_ref[0, :, 0:1] + deg_ref[1, :, 0:1] + 1.0)


def _tc1_body(x_ref, w_ref, deg_ref, o_ref):
    dinv = _dinv(deg_ref)
    o_ref[...] = jnp.dot(x_ref[...], w_ref[...],
                         preferred_element_type=jnp.float32) * dinv


def _tc2_body(a_ref, h_ref, deg_ref, b_ref, w_ref, o_ref):
    dinv = _dinv(deg_ref)
    z = jnp.maximum(dinv * (a_ref[0] + a_ref[1] + h_ref[...]) + b_ref[...],
                    0.0)
    o_ref[...] = jnp.dot(z, w_ref[...],
                         preferred_element_type=jnp.float32) * dinv


def _tc3_body(a_ref, h_ref, deg_ref, b_ref, o_ref):
    dinv = _dinv(deg_ref)
    agg = (a_ref[0] + a_ref[1] + h_ref[...])[:, :64]
    o_ref[...] = jnp.maximum(dinv * agg + b_ref[...], 0.0)


def _rows_spec(fw):
    return pl.BlockSpec((_ROWS_B, fw), lambda i: (i, 0))


def _part_spec(fw):
    return pl.BlockSpec((NC, _ROWS_B, fw), lambda i: (0, i, 0))


def _full_spec(a, b):
    return pl.BlockSpec((a, b), lambda i: (0, 0))


def kernel(x, edge_index, W1, b1, W2, b2):
    s = edge_index[0].astype(jnp.int32)
    d = edge_index[1].astype(jnp.int32)
    pad = EP - E
    padrows = N + (jnp.arange(pad, dtype=jnp.int32) % (NP - N))
    s2 = jnp.concatenate([s, padrows]).reshape(NCHUNKS, CHUNK)
    d2 = jnp.concatenate([d, padrows]).reshape(NCHUNKS, CHUNK)
    sd2 = jnp.stack([s2, d2], axis=1)
    xp = jnp.pad(x, ((0, NP - N), (0, 0)))
    W2p = jnp.pad(W2, ((0, 0), (0, 64)))
    ones128 = jnp.ones((CHUNK, 128), jnp.float32)
    z128 = jnp.zeros((CHUNK, 128), jnp.float32)

    degp = _sc_degree()(sd2, ones128, z128)

    h1s = pl.pallas_call(
        _tc1_body,
        grid=(_GRID,),
        in_specs=[_rows_spec(128), _full_spec(128, 128), _part_spec(128)],
        out_specs=_rows_spec(128),
        out_shape=jax.ShapeDtypeStruct((NP, 128), jnp.float32),
    )(xp, W1, degp)

    sc_scatter = _sc_scatter()
    a1 = sc_scatter(h1s, sd2, z128)

    h2s = pl.pallas_call(
        _tc2_body,
        grid=(_GRID,),
        in_specs=[_part_spec(128), _rows_spec(128), _part_spec(128),
                  _full_spec(1, 128), _full_spec(128, 128)],
        out_specs=_rows_spec(128),
        out_shape=jax.ShapeDtypeStruct((NP, 128), jnp.float32),
    )(a1, h1s, degp, b1.reshape(1, 128), W2p)

    a2 = sc_scatter(h2s, sd2, z128)

    outp = pl.pallas_call(
        _tc3_body,
        grid=(_GRID,),
        in_specs=[_part_spec(128), _rows_spec(128), _part_spec(128),
                  _full_spec(1, 64)],
        out_specs=_rows_spec(64),
        out_shape=jax.ShapeDtypeStruct((NP, 64), jnp.float32),
    )(a2, h2s, degp, b2.reshape(1, 64))

    return outp[:N]
